# baseline (speedup 1.0000x reference)
import functools
import math

import jax
import jax.numpy as jnp
from jax import lax
from jax.experimental import pallas as pl
from jax.experimental.pallas import tpu as pltpu
from jax.experimental.pallas import tpu_sc as plsc

D_MODEL = 768
SEQ = 4096
BATCH = 4
TOKENS = BATCH * SEQ
SCALE = math.sqrt(float(D_MODEL))

_INFO = plsc.get_sparse_core_info()
NUM_WORKERS = _INFO.num_cores * _INFO.num_subcores
TPW = TOKENS // NUM_WORKERS
CB = 16
NCHUNK = TPW // CB
VPR = D_MODEL // 16
NBUF = 3


def _pe_table(seq_len, d_model):
    pos = jnp.arange(seq_len, dtype=jnp.float32)[:, None]
    div = jnp.exp(
        jnp.arange(0, d_model, 2, dtype=jnp.float32)
        * (-jnp.log(10000.0) / d_model)
    )
    pe = jnp.zeros((seq_len, d_model), dtype=jnp.float32)
    pe = pe.at[:, 0::2].set(jnp.sin(pos * div))
    pe = pe.at[:, 1::2].set(jnp.cos(pos * div))
    return pe


def _sc_body(idx_hbm, table_hbm, pe_hbm, out_hbm, idx_v, *scratch):
    rows = scratch[0:NBUF]
    pes = scratch[NBUF:2 * NBUF]
    gsem = scratch[2 * NBUF:3 * NBUF]
    psem = scratch[3 * NBUF:4 * NBUF]
    ssem = scratch[4 * NBUF:5 * NBUF]

    wid = lax.axis_index("s") * _INFO.num_cores + lax.axis_index("c")
    base = wid * TPW
    pos0 = (wid % (SEQ // TPW)) * TPW
    pltpu.sync_copy(idx_hbm.at[pl.ds(base, TPW)], idx_v)

    gather = {}
    peload = {}
    store = {}

    def issue_gather(n):
        if n < NCHUNK:
            gather[n] = pltpu.async_copy(
                table_hbm.at[idx_v.at[pl.ds(n * CB, CB)]],
                rows[n % NBUF], gsem[n % NBUF])

    def issue_pe(n):
        if n < NCHUNK:
            if n >= NBUF:
                store[n - NBUF].wait()
            peload[n] = pltpu.async_copy(
                pe_hbm.at[pl.ds(pos0 + n * CB, CB)],
                pes[n % NBUF], psem[n % NBUF])

    issue_pe(0)
    issue_pe(1)
    issue_gather(0)
    issue_gather(1)
    issue_gather(2)

    for c in range(NCHUNK):
        issue_pe(c + 2)
        gather[c].wait()
        peload[c].wait()
        rbuf = rows[c % NBUF]
        pbuf = pes[c % NBUF]

        def fma_row(i, carry):
            for j in range(VPR):
                sl = pl.ds(j * 16, 16)
                pbuf[i, sl] = rbuf[i, sl] * SCALE + pbuf[i, sl]
            return carry

        lax.fori_loop(0, CB, fma_row, 0)
        store[c] = pltpu.async_copy(
            pbuf, out_hbm.at[pl.ds(base + c * CB, CB)], ssem[c % NBUF])
        issue_gather(c + NBUF)

    for c in range(NCHUNK - NBUF, NCHUNK):
        store[c].wait()


def kernel(x, table):
    idx = x.reshape(-1).astype(jnp.int32)
    pe = _pe_table(SEQ, D_MODEL)
    mesh = plsc.VectorSubcoreMesh(core_axis_name="c", subcore_axis_name="s")
    scratch = (
        [pltpu.VMEM((TPW,), jnp.int32)]
        + [pltpu.VMEM((CB, D_MODEL), jnp.float32) for _ in range(2 * NBUF)]
        + [pltpu.SemaphoreType.DMA for _ in range(3 * NBUF)]
    )
    run = functools.partial(
        pl.kernel,
        out_type=jax.ShapeDtypeStruct((TOKENS, D_MODEL), jnp.float32),
        mesh=mesh,
        scratch_types=scratch,
    )(_sc_body)
    out = run(idx, table, pe)
    return out.reshape(BATCH, SEQ, D_MODEL)
